# baseline (speedup 1.0000x reference)
import functools

import jax
import jax.numpy as jnp
from jax import lax
from jax.experimental import pallas as pl
from jax.experimental.pallas import tpu as pltpu, tpu_sc as plsc

_CHUNK = 128


def _build(B, V, D):
    info = plsc.get_sparse_core_info()
    nc = info.num_cores
    nw = nc * info.num_subcores
    b_per_w = B // nw
    n_chunks = b_per_w // _CHUNK
    mesh = plsc.VectorSubcoreMesh(core_axis_name="c", subcore_axis_name="s")

    @functools.partial(
        pl.kernel,
        mesh=mesh,
        out_type=jax.ShapeDtypeStruct((D, B), jnp.float32),
        compiler_params=pltpu.CompilerParams(use_tc_tiling_on_sc=False),
        scratch_types=[
            pltpu.VMEM((n_chunks, _CHUNK), jnp.int32),
            pltpu.VMEM((D, b_per_w), jnp.float32),
            pltpu.SemaphoreType.DMA,
        ],
    )
    def gather_kernel(idx_hbm, table_hbm, out_hbm, idx_v, rows_v, sem):
        wid = lax.axis_index("s") * nc + lax.axis_index("c")
        base = wid * b_per_w
        pltpu.sync_copy(idx_hbm.at[wid], idx_v)

        def fire(c, carry):
            for j in range(n_chunks):
                pltpu.async_copy(
                    table_hbm.at[c].at[idx_v.at[j]],
                    rows_v.at[c, pl.ds(j * _CHUNK, _CHUNK)],
                    sem,
                )
            return carry

        lax.fori_loop(0, D, fire, 0)
        pltpu.make_async_copy(
            table_hbm.at[:, pl.ds(0, b_per_w)], rows_v, sem
        ).wait()
        pltpu.sync_copy(rows_v, out_hbm.at[:, pl.ds(base, b_per_w)])

    return nw, n_chunks, gather_kernel


def kernel(y, table):
    B, = y.shape
    V, D = table.shape
    nw, n_chunks, gather_kernel = _build(B, V, D)
    idx3 = y.astype(jnp.int32).reshape(nw, n_chunks, _CHUNK)
    return gather_kernel(idx3, table.T).T


# speedup vs baseline: 8.0201x; 8.0201x over previous
import functools

import jax
import jax.numpy as jnp
from jax import lax
from jax.experimental import pallas as pl
from jax.experimental.pallas import tpu as pltpu, tpu_sc as plsc

_CHUNK = 128


def _build(B, V, D):
    info = plsc.get_sparse_core_info()
    nc = info.num_cores
    nw = nc * info.num_subcores
    b_per_w = B // nw
    n_chunks = b_per_w // _CHUNK
    mesh = plsc.VectorSubcoreMesh(core_axis_name="c", subcore_axis_name="s")

    @functools.partial(
        pl.kernel,
        mesh=mesh,
        out_type=jax.ShapeDtypeStruct((B, D), jnp.float32),
        compiler_params=pltpu.CompilerParams(use_tc_tiling_on_sc=False),
        scratch_types=[
            pltpu.VMEM((b_per_w,), jnp.int32),
            pltpu.VMEM((b_per_w, D), jnp.float32),
            pltpu.SemaphoreType.DMA,
        ],
    )
    def gather_kernel(idx_hbm, table_hbm, out_hbm, idx_v, rows_v, sem):
        wid = lax.axis_index("s") * nc + lax.axis_index("c")
        base = wid * b_per_w
        pltpu.sync_copy(idx_hbm.at[pl.ds(base, b_per_w)], idx_v)
        copies = [
            pltpu.async_copy(
                table_hbm.at[idx_v.at[pl.ds(j * _CHUNK, _CHUNK)]],
                rows_v.at[pl.ds(j * _CHUNK, _CHUNK)],
                sem,
            )
            for j in range(n_chunks)
        ]
        for c in copies:
            c.wait()
        pltpu.sync_copy(rows_v, out_hbm.at[pl.ds(base, b_per_w)])

    return gather_kernel


def kernel(y, table):
    B, = y.shape
    V, D = table.shape
    gather_kernel = _build(B, V, D)
    return gather_kernel(y.astype(jnp.int32), table)


# speedup vs baseline: 8.0244x; 1.0005x over previous
import functools

import jax
import jax.numpy as jnp
from jax import lax
from jax.experimental import pallas as pl
from jax.experimental.pallas import tpu as pltpu, tpu_sc as plsc

_CHUNK = 128


def _build(B, V, D):
    info = plsc.get_sparse_core_info()
    nc = info.num_cores
    nw = nc * info.num_subcores
    b_per_w = B // nw
    n_chunks = b_per_w // _CHUNK
    mesh = plsc.VectorSubcoreMesh(core_axis_name="c", subcore_axis_name="s")

    @functools.partial(
        pl.kernel,
        mesh=mesh,
        out_type=jax.ShapeDtypeStruct((B, D), jnp.float32),
        scratch_types=[
            pltpu.VMEM((b_per_w,), jnp.int32),
            pltpu.VMEM((n_chunks, _CHUNK), jnp.int32),
            pltpu.VMEM((2, _CHUNK, 2 * D), jnp.float32),
            pltpu.VMEM((b_per_w, D), jnp.float32),
            pltpu.SemaphoreType.DMA,
        ],
    )
    def gather_kernel(idx_hbm, pairs_hbm, out_hbm, idx_v, j_v, stage_v,
                      rows_v, sem):
        wid = lax.axis_index("s") * nc + lax.axis_index("c")
        base = wid * b_per_w
        pltpu.sync_copy(idx_hbm.at[pl.ds(base, b_per_w)], idx_v)

        for k in range(n_chunks):
            for g in range(_CHUNK // 16):
                rvec = idx_v[pl.ds(k * _CHUNK + g * 16, 16)]
                j_v[k, pl.ds(g * 16, 16)] = rvec >> 1

        def extract(k, buf):
            def body(g, carry):
                rvec = idx_v[pl.ds(k * _CHUNK + g * 16, 16)]
                for i in range(16):
                    b = g * 16 + i
                    off = (rvec[i] & 1) * D
                    for q in range(D // 16):
                        rows_v[k * _CHUNK + b, pl.ds(q * 16, 16)] = (
                            stage_v[buf, b, pl.ds(off + q * 16, 16)]
                        )
                return carry
            lax.fori_loop(0, _CHUNK // 16, body, 0)

        cps = {}
        for k in range(n_chunks):
            if k >= 2:
                cps[k - 2].wait()
                extract(k - 2, (k - 2) % 2)
            cps[k] = pltpu.async_copy(
                pairs_hbm.at[j_v.at[k]], stage_v.at[k % 2], sem
            )
        for k in (n_chunks - 2, n_chunks - 1):
            cps[k].wait()
            extract(k, k % 2)
        pltpu.sync_copy(rows_v, out_hbm.at[pl.ds(base, b_per_w)])

    return gather_kernel


def kernel(y, table):
    B, = y.shape
    V, D = table.shape
    gather_kernel = _build(B, V, D)
    return gather_kernel(y.astype(jnp.int32), table.reshape(V // 2, 2 * D))


# speedup vs baseline: 22.1748x; 2.7634x over previous
import functools

import jax
import jax.numpy as jnp
from jax import lax
from jax.experimental import pallas as pl
from jax.experimental.pallas import tpu as pltpu, tpu_sc as plsc

_WAVE = 256
_RING = 16


def _build(B, V, D):
    info = plsc.get_sparse_core_info()
    nc = info.num_cores
    nw = nc * info.num_subcores
    n_waves = (V // 128) * 128 // _WAVE
    waves_per_w = (n_waves + nw - 1) // nw
    cut = n_waves * _WAVE
    mesh = plsc.VectorSubcoreMesh(core_axis_name="c", subcore_axis_name="s")

    @functools.partial(
        pl.kernel,
        mesh=mesh,
        out_type=jax.ShapeDtypeStruct((B * D,), jnp.float32),
        compiler_params=pltpu.CompilerParams(needs_layout_passes=False),
        scratch_types=[
            pltpu.VMEM((B,), jnp.int32),
            pltpu.VMEM((B + 16,), jnp.int32),
            pltpu.VMEM((B + 16,), jnp.int32),
            pltpu.VMEM((2, D, _WAVE), jnp.float32),
            pltpu.VMEM((16,), jnp.int32),
            pltpu.VMEM((16,), jnp.int32),
            pltpu.VMEM((_RING, D), jnp.float32),
            pltpu.SemaphoreType.DMA,
            pltpu.SemaphoreType.DMA,
        ],
    )
    def gather_kernel(idx_hbm, tt_hbm, out_hbm, y_v, selr_v, selp_v, wave_v,
                      mr_v, mp_v, ring_v, sem_w, sem_o):
        wid = lax.axis_index("s") * nc + lax.axis_index("c")
        pltpu.sync_copy(idx_hbm, y_v)
        cvec = lax.iota(jnp.int32, 16)

        def filt(g, cnt):
            rvec = y_v[pl.ds(g * 16, 16)]
            mask = jnp.logical_and(
                ((rvec >> 8) & (nw - 1)) == wid, rvec < cut
            )
            pref = plsc.cumsum(mask.astype(jnp.int32))
            pos = cnt + pref - 1
            plsc.store_scatter(selr_v.at[:], [pos], rvec, mask=mask)
            plsc.store_scatter(selp_v.at[:], [pos], cvec + g * 16, mask=mask)
            return cnt + pref[15]

        cnt = lax.fori_loop(0, B // 16, filt, jnp.int32(0))
        plsc.store_scatter(
            selr_v.at[:], [cnt + cvec], jnp.full((16,), -1, jnp.int32),
            mask=cvec < 16,
        )
        n_chunks = (cnt + 15) >> 4

        def fire(i):
            wv = wid + i * nw

            @pl.when(jnp.logical_and(i < waves_per_w, wv < n_waves))
            def _():
                pltpu.async_copy(
                    tt_hbm.at[:, pl.ds(wv * _WAVE, _WAVE)],
                    wave_v.at[i % 2],
                    sem_w,
                )

        fire(jnp.int32(0))

        def wave_loop(i, oc):
            fire(i + 1)
            wv = wid + i * nw

            @pl.when(wv < n_waves)
            def _():
                pltpu.make_async_copy(
                    tt_hbm.at[:, pl.ds(0, _WAVE)], wave_v.at[i % 2], sem_w
                ).wait()

            def scan(c2, oc2):
                rvec = selr_v[pl.ds(c2 * 16, 16)]
                mask = (rvec >> 8) == wv
                pref = plsc.cumsum(mask.astype(jnp.int32))
                mpos = pref - 1
                plsc.store_scatter(mr_v.at[:], [mpos], rvec, mask=mask)
                plsc.store_scatter(
                    mp_v.at[:], [mpos], selp_v[pl.ds(c2 * 16, 16)], mask=mask
                )
                nm = pref[15]

                def extract(mi, oc3):
                    slot = oc3 % _RING

                    @pl.when(oc3 >= _RING)
                    def _():
                        pltpu.make_async_copy(
                            ring_v.at[slot], out_hbm.at[pl.ds(0, D)],
                            sem_o,
                        ).wait()

                    mivec = jnp.full((16,), mi, jnp.int32)
                    lane = plsc.load_gather(mr_v.at[:], [mivec]) & (_WAVE - 1)
                    for q in range(D // 16):
                        ring_v[slot, pl.ds(q * 16, 16)] = plsc.load_gather(
                            wave_v.at[i % 2], [cvec + q * 16, lane]
                        )
                    p = plsc.load_gather(mp_v.at[:], [mivec])[0]
                    off = pl.multiple_of(p * D, D)
                    pltpu.async_copy(
                        ring_v.at[slot], out_hbm.at[pl.ds(off, D)],
                        sem_o,
                    )
                    return oc3 + 1

                return lax.fori_loop(0, nm, extract, oc2)

            return lax.fori_loop(0, n_chunks, scan, oc)

        oc = lax.fori_loop(0, waves_per_w, wave_loop, jnp.int32(0))

        def drain(j, carry):
            pltpu.make_async_copy(
                ring_v.at[j], out_hbm.at[pl.ds(0, D)], sem_o
            ).wait()
            return carry

        lax.fori_loop(0, jnp.minimum(oc, _RING), drain, 0)

    return cut, gather_kernel


def kernel(y, table):
    B, = y.shape
    V, D = table.shape
    cut, gather_kernel = _build(B, V, D)
    yi = y.astype(jnp.int32)
    flat = gather_kernel(yi, table.T)
    out = flat.reshape(B, D)
    tail_tab = table[cut:, :]
    tail = jnp.take(tail_tab, jnp.clip(yi - cut, 0, V - cut - 1), axis=0)
    return jnp.where((yi >= cut)[:, None], tail, out)


# speedup vs baseline: 27.4383x; 1.2374x over previous
import functools

import jax
import jax.numpy as jnp
from jax import lax
from jax.experimental import pallas as pl
from jax.experimental.pallas import tpu as pltpu, tpu_sc as plsc

_WAVE = 512
_RING = 16


def _build(B, V, D):
    info = plsc.get_sparse_core_info()
    nc = info.num_cores
    nw = nc * info.num_subcores
    n_waves = (V // 128) * 128 // _WAVE
    waves_per_w = (n_waves + nw - 1) // nw
    cut = n_waves * _WAVE
    mesh = plsc.VectorSubcoreMesh(core_axis_name="c", subcore_axis_name="s")

    @functools.partial(
        pl.kernel,
        mesh=mesh,
        out_type=jax.ShapeDtypeStruct((B * D,), jnp.float32),
        compiler_params=pltpu.CompilerParams(needs_layout_passes=False),
        scratch_types=[
            pltpu.VMEM((B,), jnp.int32),
            pltpu.VMEM((B + 16,), jnp.int32),
            pltpu.VMEM((B + 16,), jnp.int32),
            pltpu.VMEM((2, D, _WAVE), jnp.float32),
            pltpu.VMEM((16,), jnp.int32),
            pltpu.VMEM((16,), jnp.int32),
            pltpu.VMEM((_RING, D), jnp.float32),
            pltpu.SemaphoreType.DMA,
            pltpu.SemaphoreType.DMA,
        ],
    )
    def gather_kernel(idx_hbm, tt_hbm, out_hbm, y_v, selr_v, selp_v, wave_v,
                      mr_v, mp_v, ring_v, sem_w, sem_o):
        wid = lax.axis_index("s") * nc + lax.axis_index("c")
        pltpu.sync_copy(idx_hbm, y_v)
        cvec = lax.iota(jnp.int32, 16)

        shift = _WAVE.bit_length() - 1

        def filt(g, cnt):
            rvec = y_v[pl.ds(g * 16, 16)]
            mask = jnp.logical_and(
                ((rvec >> shift) & (nw - 1)) == wid, rvec < cut
            )
            nm = plsc.all_reduce_population_count(mask)[0]

            @pl.when(nm > 0)
            def _():
                pos = cnt + plsc.cumsum(mask.astype(jnp.int32)) - 1
                plsc.store_scatter(selr_v.at[:], [pos], rvec, mask=mask)
                plsc.store_scatter(
                    selp_v.at[:], [pos], cvec + g * 16, mask=mask
                )

            return cnt + nm

        cnt = lax.fori_loop(0, B // 16, filt, jnp.int32(0))
        plsc.store_scatter(
            selr_v.at[:], [cnt + cvec], jnp.full((16,), -1, jnp.int32),
            mask=cvec < 16,
        )
        n_chunks = (cnt + 15) >> 4

        def fire(i):
            wv = wid + i * nw

            @pl.when(jnp.logical_and(i < waves_per_w, wv < n_waves))
            def _():
                pltpu.async_copy(
                    tt_hbm.at[:, pl.ds(wv * _WAVE, _WAVE)],
                    wave_v.at[i % 2],
                    sem_w,
                )

        fire(jnp.int32(0))

        def wave_loop(i, oc):
            fire(i + 1)
            wv = wid + i * nw

            @pl.when(wv < n_waves)
            def _():
                pltpu.make_async_copy(
                    tt_hbm.at[:, pl.ds(0, _WAVE)], wave_v.at[i % 2], sem_w
                ).wait()

            def scan(c2, oc2):
                rvec = selr_v[pl.ds(c2 * 16, 16)]
                mask = (rvec >> shift) == wv
                nm = plsc.all_reduce_population_count(mask)[0]

                @pl.when(nm > 0)
                def _():
                    mpos = plsc.cumsum(mask.astype(jnp.int32)) - 1
                    plsc.store_scatter(mr_v.at[:], [mpos], rvec, mask=mask)
                    plsc.store_scatter(
                        mp_v.at[:], [mpos], selp_v[pl.ds(c2 * 16, 16)],
                        mask=mask,
                    )

                def extract(mi, oc3):
                    slot = oc3 % _RING

                    @pl.when(oc3 >= _RING)
                    def _():
                        pltpu.make_async_copy(
                            ring_v.at[slot], out_hbm.at[pl.ds(0, D)],
                            sem_o,
                        ).wait()

                    mivec = jnp.full((16,), mi, jnp.int32)
                    lane = plsc.load_gather(mr_v.at[:], [mivec]) & (_WAVE - 1)
                    for q in range(D // 16):
                        ring_v[slot, pl.ds(q * 16, 16)] = plsc.load_gather(
                            wave_v.at[i % 2], [cvec + q * 16, lane]
                        )
                    p = plsc.load_gather(mp_v.at[:], [mivec])[0]
                    off = pl.multiple_of(p * D, D)
                    pltpu.async_copy(
                        ring_v.at[slot], out_hbm.at[pl.ds(off, D)],
                        sem_o,
                    )
                    return oc3 + 1

                return lax.fori_loop(0, nm, extract, oc2)

            return lax.fori_loop(0, n_chunks, scan, oc)

        oc = lax.fori_loop(0, waves_per_w, wave_loop, jnp.int32(0))

        def drain(j, carry):
            pltpu.make_async_copy(
                ring_v.at[j], out_hbm.at[pl.ds(0, D)], sem_o
            ).wait()
            return carry

        lax.fori_loop(0, jnp.minimum(oc, _RING), drain, 0)

    return cut, gather_kernel


def kernel(y, table):
    B, = y.shape
    V, D = table.shape
    cut, gather_kernel = _build(B, V, D)
    yi = y.astype(jnp.int32)
    flat = gather_kernel(yi, table.T)
    out = flat.reshape(B, D)
    tail_tab = table[cut:, :]
    tail = jnp.take(tail_tab, jnp.clip(yi - cut, 0, V - cut - 1), axis=0)
    return jnp.where((yi >= cut)[:, None], tail, out)


# speedup vs baseline: 28.9849x; 1.0564x over previous
import functools

import jax
import jax.numpy as jnp
from jax import lax
from jax.experimental import pallas as pl
from jax.experimental.pallas import tpu as pltpu, tpu_sc as plsc

_WAVE = 512
_RING = 16


def _build(B, V, D):
    info = plsc.get_sparse_core_info()
    nc = info.num_cores
    nw = nc * info.num_subcores
    n_waves = (V // 128) * 128 // _WAVE
    waves_per_w = (n_waves + nw - 1) // nw
    cut = n_waves * _WAVE
    mesh = plsc.VectorSubcoreMesh(core_axis_name="c", subcore_axis_name="s")

    @functools.partial(
        pl.kernel,
        mesh=mesh,
        out_type=jax.ShapeDtypeStruct((B * D,), jnp.float32),
        compiler_params=pltpu.CompilerParams(needs_layout_passes=False),
        scratch_types=[
            pltpu.VMEM((B,), jnp.int32),
            pltpu.VMEM((B + 32,), jnp.int32),
            pltpu.VMEM((B + 32,), jnp.int32),
            pltpu.VMEM((2, D, _WAVE), jnp.float32),
            pltpu.VMEM((32,), jnp.int32),
            pltpu.VMEM((32,), jnp.int32),
            pltpu.VMEM((_RING, D), jnp.float32),
            pltpu.SemaphoreType.DMA,
            pltpu.SemaphoreType.DMA,
        ],
    )
    def gather_kernel(idx_hbm, tt_hbm, out_hbm, y_v, selr_v, selp_v, wave_v,
                      mr_v, mp_v, ring_v, sem_w, sem_o):
        wid = lax.axis_index("s") * nc + lax.axis_index("c")
        pltpu.sync_copy(idx_hbm, y_v)
        cvec = lax.iota(jnp.int32, 16)

        shift = _WAVE.bit_length() - 1

        def filt(g, cnt):
            rvec = y_v[pl.ds(g * 16, 16)]
            mask = jnp.logical_and(
                ((rvec >> shift) & (nw - 1)) == wid, rvec < cut
            )
            nm = plsc.all_reduce_population_count(mask)[0]

            @pl.when(nm > 0)
            def _():
                pos = cnt + plsc.cumsum(mask.astype(jnp.int32)) - 1
                plsc.store_scatter(selr_v.at[:], [pos], rvec, mask=mask)
                plsc.store_scatter(
                    selp_v.at[:], [pos], cvec + g * 16, mask=mask
                )

            return cnt + nm

        cnt = lax.fori_loop(0, B // 16, filt, jnp.int32(0))
        sent = jnp.full((16,), -1, jnp.int32)
        plsc.store_scatter(selr_v.at[:], [cnt + cvec], sent, mask=cvec < 16)
        plsc.store_scatter(
            selr_v.at[:], [cnt + 16 + cvec], sent, mask=cvec < 16
        )
        n_pairs = (cnt + 31) >> 5

        def fire(i):
            wv = wid + i * nw

            @pl.when(jnp.logical_and(i < waves_per_w, wv < n_waves))
            def _():
                pltpu.async_copy(
                    tt_hbm.at[:, pl.ds(wv * _WAVE, _WAVE)],
                    wave_v.at[i % 2],
                    sem_w,
                )

        fire(jnp.int32(0))

        def wave_loop(i, oc):
            fire(i + 1)
            wv = wid + i * nw

            @pl.when(wv < n_waves)
            def _():
                pltpu.make_async_copy(
                    tt_hbm.at[:, pl.ds(0, _WAVE)], wave_v.at[i % 2], sem_w
                ).wait()

            def scan(c4, oc2):
                rvec = selr_v[pl.ds(c4 * 32, 16)]
                rvec2 = selr_v[pl.ds(c4 * 32 + 16, 16)]
                mask = (rvec >> shift) == wv
                mask2 = (rvec2 >> shift) == wv
                nm1 = plsc.all_reduce_population_count(mask)[0]
                nm2 = plsc.all_reduce_population_count(mask2)[0]
                nm = nm1 + nm2

                @pl.when(nm > 0)
                def _():
                    mpos = plsc.cumsum(mask.astype(jnp.int32)) - 1
                    plsc.store_scatter(mr_v.at[:], [mpos], rvec, mask=mask)
                    plsc.store_scatter(
                        mp_v.at[:], [mpos], selp_v[pl.ds(c4 * 32, 16)],
                        mask=mask,
                    )
                    mpos2 = nm1 + plsc.cumsum(mask2.astype(jnp.int32)) - 1
                    plsc.store_scatter(mr_v.at[:], [mpos2], rvec2, mask=mask2)
                    plsc.store_scatter(
                        mp_v.at[:], [mpos2], selp_v[pl.ds(c4 * 32 + 16, 16)],
                        mask=mask2,
                    )

                def extract(mi, oc3):
                    slot = oc3 % _RING

                    @pl.when(oc3 >= _RING)
                    def _():
                        pltpu.make_async_copy(
                            ring_v.at[slot], out_hbm.at[pl.ds(0, D)],
                            sem_o,
                        ).wait()

                    mivec = jnp.full((16,), mi, jnp.int32)
                    lane = plsc.load_gather(mr_v.at[:], [mivec]) & (_WAVE - 1)
                    for q in range(D // 16):
                        ring_v[slot, pl.ds(q * 16, 16)] = plsc.load_gather(
                            wave_v.at[i % 2], [cvec + q * 16, lane]
                        )
                    p = plsc.load_gather(mp_v.at[:], [mivec])[0]
                    off = pl.multiple_of(p * D, D)
                    pltpu.async_copy(
                        ring_v.at[slot], out_hbm.at[pl.ds(off, D)],
                        sem_o,
                    )
                    return oc3 + 1

                return lax.fori_loop(0, nm, extract, oc2)

            return lax.fori_loop(0, n_pairs, scan, oc)

        oc = lax.fori_loop(0, waves_per_w, wave_loop, jnp.int32(0))

        def drain(j, carry):
            pltpu.make_async_copy(
                ring_v.at[j], out_hbm.at[pl.ds(0, D)], sem_o
            ).wait()
            return carry

        lax.fori_loop(0, jnp.minimum(oc, _RING), drain, 0)

    return cut, gather_kernel


def kernel(y, table):
    B, = y.shape
    V, D = table.shape
    cut, gather_kernel = _build(B, V, D)
    yi = y.astype(jnp.int32)
    flat = gather_kernel(yi, table.T)
    out = flat.reshape(B, D)
    tail_tab = table[cut:, :]
    tail = jnp.take(tail_tab, jnp.clip(yi - cut, 0, V - cut - 1), axis=0)
    return jnp.where((yi >= cut)[:, None], tail, out)


# speedup vs baseline: 31.0312x; 1.0706x over previous
import functools

import jax
import jax.numpy as jnp
from jax import lax
from jax.experimental import pallas as pl
from jax.experimental.pallas import tpu as pltpu, tpu_sc as plsc

_WAVE = 512
_RING = 16


def _build(B, V, D):
    info = plsc.get_sparse_core_info()
    nc = info.num_cores
    nw = nc * info.num_subcores
    n_waves = (V // 128) * 128 // _WAVE
    waves_per_w = (n_waves + nw - 1) // nw
    cut = n_waves * _WAVE
    mesh = plsc.VectorSubcoreMesh(core_axis_name="c", subcore_axis_name="s")

    @functools.partial(
        pl.kernel,
        mesh=mesh,
        out_type=jax.ShapeDtypeStruct((B * D,), jnp.float32),
        compiler_params=pltpu.CompilerParams(needs_layout_passes=False),
        scratch_types=[
            pltpu.VMEM((B,), jnp.int32),
            pltpu.VMEM((B + 32,), jnp.int32),
            pltpu.VMEM((B + 32,), jnp.int32),
            pltpu.VMEM((2, D, _WAVE), jnp.float32),
            pltpu.VMEM((32,), jnp.int32),
            pltpu.VMEM((32,), jnp.int32),
            pltpu.VMEM((_RING, D), jnp.float32),
            pltpu.SemaphoreType.DMA,
            pltpu.SemaphoreType.DMA,
        ],
    )
    def gather_kernel(idx_hbm, tt_hbm, out_hbm, y_v, selr_v, selp_v, wave_v,
                      mr_v, mp_v, ring_v, sem_w, sem_o):
        wid = lax.axis_index("s") * nc + lax.axis_index("c")
        pltpu.sync_copy(idx_hbm, y_v)
        cvec = lax.iota(jnp.int32, 16)

        shift = _WAVE.bit_length() - 1

        def filt(g, cnt):
            rvec = y_v[pl.ds(g * 32, 16)]
            rvec2 = y_v[pl.ds(g * 32 + 16, 16)]
            mask = jnp.logical_and(
                ((rvec >> shift) & (nw - 1)) == wid, rvec < cut
            )
            mask2 = jnp.logical_and(
                ((rvec2 >> shift) & (nw - 1)) == wid, rvec2 < cut
            )
            nm1 = plsc.all_reduce_population_count(mask)[0]
            nm2 = plsc.all_reduce_population_count(mask2)[0]
            nm = nm1 + nm2

            @pl.when(nm > 0)
            def _():
                pos = cnt + plsc.cumsum(mask.astype(jnp.int32)) - 1
                plsc.store_scatter(selr_v.at[:], [pos], rvec, mask=mask)
                plsc.store_scatter(
                    selp_v.at[:], [pos], cvec + g * 32, mask=mask
                )
                pos2 = cnt + nm1 + plsc.cumsum(mask2.astype(jnp.int32)) - 1
                plsc.store_scatter(selr_v.at[:], [pos2], rvec2, mask=mask2)
                plsc.store_scatter(
                    selp_v.at[:], [pos2], cvec + g * 32 + 16, mask=mask2
                )

            return cnt + nm

        cnt = lax.fori_loop(0, B // 32, filt, jnp.int32(0))
        sent = jnp.full((16,), -1, jnp.int32)
        plsc.store_scatter(selr_v.at[:], [cnt + cvec], sent, mask=cvec < 16)
        plsc.store_scatter(
            selr_v.at[:], [cnt + 16 + cvec], sent, mask=cvec < 16
        )
        n_pairs = (cnt + 31) >> 5

        def fire(i):
            wv = wid + i * nw

            @pl.when(jnp.logical_and(i < waves_per_w, wv < n_waves))
            def _():
                pltpu.async_copy(
                    tt_hbm.at[:, pl.ds(wv * _WAVE, _WAVE)],
                    wave_v.at[i % 2],
                    sem_w,
                )

        fire(jnp.int32(0))

        def wave_loop(i, oc):
            fire(i + 1)
            wv = wid + i * nw

            @pl.when(wv < n_waves)
            def _():
                pltpu.make_async_copy(
                    tt_hbm.at[:, pl.ds(0, _WAVE)], wave_v.at[i % 2], sem_w
                ).wait()

            def scan(c4, oc2):
                rvec = selr_v[pl.ds(c4 * 32, 16)]
                rvec2 = selr_v[pl.ds(c4 * 32 + 16, 16)]
                mask = (rvec >> shift) == wv
                mask2 = (rvec2 >> shift) == wv
                nm1 = plsc.all_reduce_population_count(mask)[0]
                nm2 = plsc.all_reduce_population_count(mask2)[0]
                nm = nm1 + nm2

                @pl.when(nm > 0)
                def _():
                    mpos = plsc.cumsum(mask.astype(jnp.int32)) - 1
                    plsc.store_scatter(mr_v.at[:], [mpos], rvec, mask=mask)
                    plsc.store_scatter(
                        mp_v.at[:], [mpos], selp_v[pl.ds(c4 * 32, 16)],
                        mask=mask,
                    )
                    mpos2 = nm1 + plsc.cumsum(mask2.astype(jnp.int32)) - 1
                    plsc.store_scatter(mr_v.at[:], [mpos2], rvec2, mask=mask2)
                    plsc.store_scatter(
                        mp_v.at[:], [mpos2], selp_v[pl.ds(c4 * 32 + 16, 16)],
                        mask=mask2,
                    )

                def extract(mi, oc3):
                    slot = oc3 % _RING

                    @pl.when(oc3 >= _RING)
                    def _():
                        pltpu.make_async_copy(
                            ring_v.at[slot], out_hbm.at[pl.ds(0, D)],
                            sem_o,
                        ).wait()

                    mivec = jnp.full((16,), mi, jnp.int32)
                    lane = plsc.load_gather(mr_v.at[:], [mivec]) & (_WAVE - 1)
                    for q in range(D // 16):
                        ring_v[slot, pl.ds(q * 16, 16)] = plsc.load_gather(
                            wave_v.at[i % 2], [cvec + q * 16, lane]
                        )
                    p = plsc.load_gather(mp_v.at[:], [mivec])[0]
                    off = pl.multiple_of(p * D, D)
                    pltpu.async_copy(
                        ring_v.at[slot], out_hbm.at[pl.ds(off, D)],
                        sem_o,
                    )
                    return oc3 + 1

                return lax.fori_loop(0, nm, extract, oc2)

            return lax.fori_loop(0, n_pairs, scan, oc)

        oc = lax.fori_loop(0, waves_per_w, wave_loop, jnp.int32(0))

        def drain(j, carry):
            pltpu.make_async_copy(
                ring_v.at[j], out_hbm.at[pl.ds(0, D)], sem_o
            ).wait()
            return carry

        lax.fori_loop(0, jnp.minimum(oc, _RING), drain, 0)

    return cut, gather_kernel


def kernel(y, table):
    B, = y.shape
    V, D = table.shape
    cut, gather_kernel = _build(B, V, D)
    yi = y.astype(jnp.int32)
    flat = gather_kernel(yi, table.T)
    out = flat.reshape(B, D)
    tail_tab = table[cut:, :]
    tail = jnp.take(tail_tab, jnp.clip(yi - cut, 0, V - cut - 1), axis=0)
    return jnp.where((yi >= cut)[:, None], tail, out)


# speedup vs baseline: 31.7638x; 1.0236x over previous
import functools

import jax
import jax.numpy as jnp
from jax import lax
from jax.experimental import pallas as pl
from jax.experimental.pallas import tpu as pltpu, tpu_sc as plsc

_WAVE = 512
_RING = 16


def _build(B, V, D):
    info = plsc.get_sparse_core_info()
    nc = info.num_cores
    nw = nc * info.num_subcores
    n_waves = (V // 128) * 128 // _WAVE
    waves_per_w = (n_waves + nw - 1) // nw
    cut = n_waves * _WAVE
    mesh = plsc.VectorSubcoreMesh(core_axis_name="c", subcore_axis_name="s")

    @functools.partial(
        pl.kernel,
        mesh=mesh,
        out_type=jax.ShapeDtypeStruct((B * D,), jnp.float32),
        compiler_params=pltpu.CompilerParams(needs_layout_passes=False),
        scratch_types=[
            pltpu.VMEM((B,), jnp.int32),
            pltpu.VMEM((B + 64,), jnp.int32),
            pltpu.VMEM((B + 64,), jnp.int32),
            pltpu.VMEM((2, D, _WAVE), jnp.float32),
            pltpu.VMEM((64,), jnp.int32),
            pltpu.VMEM((64,), jnp.int32),
            pltpu.VMEM((_RING, D), jnp.float32),
            pltpu.SemaphoreType.DMA,
            pltpu.SemaphoreType.DMA,
        ],
    )
    def gather_kernel(idx_hbm, tt_hbm, out_hbm, y_v, selr_v, selp_v, wave_v,
                      mr_v, mp_v, ring_v, sem_w, sem_o):
        wid = lax.axis_index("s") * nc + lax.axis_index("c")
        pltpu.sync_copy(idx_hbm, y_v)
        cvec = lax.iota(jnp.int32, 16)

        shift = _WAVE.bit_length() - 1

        def filt(g, cnt):
            rvec = y_v[pl.ds(g * 32, 16)]
            rvec2 = y_v[pl.ds(g * 32 + 16, 16)]
            mask = jnp.logical_and(
                ((rvec >> shift) & (nw - 1)) == wid, rvec < cut
            )
            mask2 = jnp.logical_and(
                ((rvec2 >> shift) & (nw - 1)) == wid, rvec2 < cut
            )
            nm1 = plsc.all_reduce_population_count(mask)[0]
            nm2 = plsc.all_reduce_population_count(mask2)[0]
            nm = nm1 + nm2

            @pl.when(nm > 0)
            def _():
                pos = cnt + plsc.cumsum(mask.astype(jnp.int32)) - 1
                plsc.store_scatter(selr_v.at[:], [pos], rvec, mask=mask)
                plsc.store_scatter(
                    selp_v.at[:], [pos], cvec + g * 32, mask=mask
                )
                pos2 = cnt + nm1 + plsc.cumsum(mask2.astype(jnp.int32)) - 1
                plsc.store_scatter(selr_v.at[:], [pos2], rvec2, mask=mask2)
                plsc.store_scatter(
                    selp_v.at[:], [pos2], cvec + g * 32 + 16, mask=mask2
                )

            return cnt + nm

        cnt = lax.fori_loop(0, B // 32, filt, jnp.int32(0))
        sent = jnp.full((16,), -1, jnp.int32)
        for u in range(4):
            plsc.store_scatter(
                selr_v.at[:], [cnt + 16 * u + cvec], sent, mask=cvec < 16
            )
        n_pairs = (cnt + 63) >> 6

        def fire(i):
            wv = wid + i * nw

            @pl.when(jnp.logical_and(i < waves_per_w, wv < n_waves))
            def _():
                pltpu.async_copy(
                    tt_hbm.at[:, pl.ds(wv * _WAVE, _WAVE)],
                    wave_v.at[i % 2],
                    sem_w,
                )

        fire(jnp.int32(0))

        def wave_loop(i, oc):
            fire(i + 1)
            wv = wid + i * nw

            @pl.when(wv < n_waves)
            def _():
                pltpu.make_async_copy(
                    tt_hbm.at[:, pl.ds(0, _WAVE)], wave_v.at[i % 2], sem_w
                ).wait()

            def scan(c4, oc2):
                rv = [selr_v[pl.ds(c4 * 64 + 16 * u, 16)] for u in range(4)]
                ms = [(r >> shift) == wv for r in rv]
                pc = [plsc.all_reduce_population_count(m)[0] for m in ms]
                nm = pc[0] + pc[1] + pc[2] + pc[3]

                @pl.when(nm > 0)
                def _():
                    base = jnp.int32(0)
                    for u in range(4):
                        mpos = base + plsc.cumsum(ms[u].astype(jnp.int32)) - 1
                        plsc.store_scatter(
                            mr_v.at[:], [mpos], rv[u], mask=ms[u]
                        )
                        plsc.store_scatter(
                            mp_v.at[:], [mpos],
                            selp_v[pl.ds(c4 * 64 + 16 * u, 16)], mask=ms[u],
                        )
                        base = base + pc[u]

                def extract(mi, oc3):
                    slot = oc3 % _RING

                    @pl.when(oc3 >= _RING)
                    def _():
                        pltpu.make_async_copy(
                            ring_v.at[slot], out_hbm.at[pl.ds(0, D)],
                            sem_o,
                        ).wait()

                    mivec = jnp.full((16,), mi, jnp.int32)
                    lane = plsc.load_gather(mr_v.at[:], [mivec]) & (_WAVE - 1)
                    for q in range(D // 16):
                        ring_v[slot, pl.ds(q * 16, 16)] = plsc.load_gather(
                            wave_v.at[i % 2], [cvec + q * 16, lane]
                        )
                    p = plsc.load_gather(mp_v.at[:], [mivec])[0]
                    off = pl.multiple_of(p * D, D)
                    pltpu.async_copy(
                        ring_v.at[slot], out_hbm.at[pl.ds(off, D)],
                        sem_o,
                    )
                    return oc3 + 1

                return lax.fori_loop(0, nm, extract, oc2)

            return lax.fori_loop(0, n_pairs, scan, oc)

        oc = lax.fori_loop(0, waves_per_w, wave_loop, jnp.int32(0))

        def drain(j, carry):
            pltpu.make_async_copy(
                ring_v.at[j], out_hbm.at[pl.ds(0, D)], sem_o
            ).wait()
            return carry

        lax.fori_loop(0, jnp.minimum(oc, _RING), drain, 0)

    return cut, gather_kernel


def kernel(y, table):
    B, = y.shape
    V, D = table.shape
    cut, gather_kernel = _build(B, V, D)
    yi = y.astype(jnp.int32)
    flat = gather_kernel(yi, table.T)
    out = flat.reshape(B, D)
    tail_tab = table[cut:, :]
    tail = jnp.take(tail_tab, jnp.clip(yi - cut, 0, V - cut - 1), axis=0)
    return jnp.where((yi >= cut)[:, None], tail, out)


# speedup vs baseline: 33.6630x; 1.0598x over previous
import functools

import jax
import jax.numpy as jnp
from jax import lax
from jax.experimental import pallas as pl
from jax.experimental.pallas import tpu as pltpu, tpu_sc as plsc

_WAVE = 512
_RING = 16


def _build(B, V, D):
    info = plsc.get_sparse_core_info()
    nc = info.num_cores
    nw = nc * info.num_subcores
    n_waves = (V // 128) * 128 // _WAVE
    waves_per_w = (n_waves + nw - 1) // nw
    cut = n_waves * _WAVE
    mesh = plsc.VectorSubcoreMesh(core_axis_name="c", subcore_axis_name="s")

    @functools.partial(
        pl.kernel,
        mesh=mesh,
        out_type=jax.ShapeDtypeStruct((B * D,), jnp.float32),
        compiler_params=pltpu.CompilerParams(needs_layout_passes=False),
        scratch_types=[
            pltpu.VMEM((B,), jnp.int32),
            pltpu.VMEM((B + 64,), jnp.int32),
            pltpu.VMEM((B + 64,), jnp.int32),
            pltpu.VMEM((2, D, _WAVE), jnp.float32),
            pltpu.VMEM((64,), jnp.int32),
            pltpu.VMEM((64,), jnp.int32),
            pltpu.VMEM((_RING, D), jnp.float32),
            pltpu.VMEM((D, 64), jnp.float32),
            pltpu.SemaphoreType.DMA,
            pltpu.SemaphoreType.DMA,
        ],
    )
    def gather_kernel(idx_hbm, tt_hbm, tail_hbm, out_hbm, y_v, selr_v,
                      selp_v, wave_v, mr_v, mp_v, ring_v, tail_v, sem_w,
                      sem_o):
        wid = lax.axis_index("s") * nc + lax.axis_index("c")
        pltpu.sync_copy(idx_hbm, y_v)
        pltpu.sync_copy(tail_hbm, tail_v)
        cvec = lax.iota(jnp.int32, 16)

        shift = _WAVE.bit_length() - 1

        def filt(g, cnt):
            rvec = y_v[pl.ds(g * 32, 16)]
            rvec2 = y_v[pl.ds(g * 32 + 16, 16)]
            mask = ((rvec >> shift) & (nw - 1)) == wid
            mask2 = ((rvec2 >> shift) & (nw - 1)) == wid
            nm1 = plsc.all_reduce_population_count(mask)[0]
            nm2 = plsc.all_reduce_population_count(mask2)[0]
            nm = nm1 + nm2

            @pl.when(nm > 0)
            def _():
                pos = cnt + plsc.cumsum(mask.astype(jnp.int32)) - 1
                plsc.store_scatter(selr_v.at[:], [pos], rvec, mask=mask)
                plsc.store_scatter(
                    selp_v.at[:], [pos], cvec + g * 32, mask=mask
                )
                pos2 = cnt + nm1 + plsc.cumsum(mask2.astype(jnp.int32)) - 1
                plsc.store_scatter(selr_v.at[:], [pos2], rvec2, mask=mask2)
                plsc.store_scatter(
                    selp_v.at[:], [pos2], cvec + g * 32 + 16, mask=mask2
                )

            return cnt + nm

        cnt = lax.fori_loop(0, B // 32, filt, jnp.int32(0))
        sent = jnp.full((16,), -1, jnp.int32)
        for u in range(4):
            plsc.store_scatter(
                selr_v.at[:], [cnt + 16 * u + cvec], sent, mask=cvec < 16
            )
        n_pairs = (cnt + 63) >> 6

        def fire(i):
            wv = wid + i * nw

            @pl.when(jnp.logical_and(i < waves_per_w, wv < n_waves))
            def _():
                pltpu.async_copy(
                    tt_hbm.at[:, pl.ds(wv * _WAVE, _WAVE)],
                    wave_v.at[i % 2],
                    sem_w,
                )

        fire(jnp.int32(0))

        def wave_loop(i, oc):
            fire(i + 1)
            wv = wid + i * nw

            @pl.when(wv < n_waves)
            def _():
                pltpu.make_async_copy(
                    tt_hbm.at[:, pl.ds(0, _WAVE)], wave_v.at[i % 2], sem_w
                ).wait()

            def scan(c4, oc2):
                rv = [selr_v[pl.ds(c4 * 64 + 16 * u, 16)] for u in range(4)]
                ms = [(r >> shift) == wv for r in rv]
                pc = [plsc.all_reduce_population_count(m)[0] for m in ms]
                nm = pc[0] + pc[1] + pc[2] + pc[3]

                @pl.when(nm > 0)
                def _():
                    base = jnp.int32(0)
                    for u in range(4):
                        mpos = base + plsc.cumsum(ms[u].astype(jnp.int32)) - 1
                        plsc.store_scatter(
                            mr_v.at[:], [mpos], rv[u], mask=ms[u]
                        )
                        plsc.store_scatter(
                            mp_v.at[:], [mpos],
                            selp_v[pl.ds(c4 * 64 + 16 * u, 16)], mask=ms[u],
                        )
                        base = base + pc[u]

                def extract(mi, oc3):
                    slot = oc3 % _RING

                    @pl.when(oc3 >= _RING)
                    def _():
                        pltpu.make_async_copy(
                            ring_v.at[slot], out_hbm.at[pl.ds(0, D)],
                            sem_o,
                        ).wait()

                    mivec = jnp.full((16,), mi, jnp.int32)
                    lane = plsc.load_gather(mr_v.at[:], [mivec]) & (_WAVE - 1)
                    for q in range(D // 16):
                        ring_v[slot, pl.ds(q * 16, 16)] = plsc.load_gather(
                            wave_v.at[i % 2], [cvec + q * 16, lane]
                        )
                    p = plsc.load_gather(mp_v.at[:], [mivec])[0]
                    off = pl.multiple_of(p * D, D)
                    pltpu.async_copy(
                        ring_v.at[slot], out_hbm.at[pl.ds(off, D)],
                        sem_o,
                    )
                    return oc3 + 1

                return lax.fori_loop(0, nm, extract, oc2)

            return lax.fori_loop(0, n_pairs, scan, oc)

        oc = lax.fori_loop(0, waves_per_w, wave_loop, jnp.int32(0))

        def tail_scan(c2, oc2):
            rvec = selr_v[pl.ds(c2 * 16, 16)]
            mask = rvec >= cut
            nm = plsc.all_reduce_population_count(mask)[0]

            @pl.when(nm > 0)
            def _():
                mpos = plsc.cumsum(mask.astype(jnp.int32)) - 1
                plsc.store_scatter(mr_v.at[:], [mpos], rvec, mask=mask)
                plsc.store_scatter(
                    mp_v.at[:], [mpos], selp_v[pl.ds(c2 * 16, 16)],
                    mask=mask,
                )

            def textract(mi, oc3):
                slot = oc3 % _RING

                @pl.when(oc3 >= _RING)
                def _():
                    pltpu.make_async_copy(
                        ring_v.at[slot], out_hbm.at[pl.ds(0, D)], sem_o
                    ).wait()

                mivec = jnp.full((16,), mi, jnp.int32)
                lane = plsc.load_gather(mr_v.at[:], [mivec]) - cut
                for q in range(D // 16):
                    ring_v[slot, pl.ds(q * 16, 16)] = plsc.load_gather(
                        tail_v, [cvec + q * 16, lane]
                    )
                p = plsc.load_gather(mp_v.at[:], [mivec])[0]
                off = pl.multiple_of(p * D, D)
                pltpu.async_copy(
                    ring_v.at[slot], out_hbm.at[pl.ds(off, D)], sem_o
                )
                return oc3 + 1

            return lax.fori_loop(0, nm, textract, oc2)

        oc = lax.fori_loop(0, (cnt + 15) >> 4, tail_scan, oc)

        def drain(j, carry):
            pltpu.make_async_copy(
                ring_v.at[j], out_hbm.at[pl.ds(0, D)], sem_o
            ).wait()
            return carry

        lax.fori_loop(0, jnp.minimum(oc, _RING), drain, 0)

    return cut, gather_kernel


def kernel(y, table):
    B, = y.shape
    V, D = table.shape
    cut, gather_kernel = _build(B, V, D)
    yi = y.astype(jnp.int32)
    tableT = table.T
    flat = gather_kernel(yi, tableT, tableT[:, cut:])
    return flat.reshape(B, D)
